# baseline (speedup 1.0000x reference)
import functools

import jax
import jax.numpy as jnp
from jax import lax
from jax.experimental import pallas as pl
from jax.experimental.pallas import tpu as pltpu
from jax.experimental.pallas import tpu_sc as plsc

N_NODES = 10000
N_EDGES = 320000
D_IN = 128
D_OUT = 128

NC = 2
NS = 16
NW = NC * NS
LANES = 16

K = 64
EPW = 10240
NCHUNK = EPW // K
E_PAD = EPW * NW
NSETS = 4
ROWS_PER_SUB = 624
ROWS_TAIL = N_NODES - ROWS_PER_SUB * NS


def _sc_spmm(support, cols, vals, rows):
    mesh = plsc.VectorSubcoreMesh(
        core_axis_name="c", subcore_axis_name="s", num_cores=NC, num_subcores=NS
    )

    scratch = [pltpu.VMEM_SHARED((N_NODES, D_OUT), jnp.float32)]
    for _ in range(NSETS):
        scratch += [
            pltpu.VMEM((K,), jnp.int32),
            pltpu.VMEM((K,), jnp.float32),
            pltpu.VMEM((K,), jnp.int32),
            pltpu.VMEM((K, D_OUT), jnp.float32),
            pltpu.SemaphoreType.DMA,
            pltpu.SemaphoreType.DMA,
            pltpu.SemaphoreType.DMA,
        ]

    @functools.partial(
        pl.kernel,
        mesh=mesh,
        out_type=jax.ShapeDtypeStruct((NC, N_NODES, D_OUT), jnp.float32),
        scratch_types=scratch,
    )
    def spmm(support_hbm, cols_hbm, vals_hbm, rows_hbm, out_hbm, acc, *bufs):
        c = lax.axis_index("c")
        s = lax.axis_index("s")
        wid = s * NC + c
        pbase = wid * NCHUNK
        sets = tuple(bufs[i * 7:(i + 1) * 7] for i in range(NSETS))
        zbuf = sets[NSETS - 1][3]

        def idx_copies(k, st):
            sl = pl.ds((pbase + k) * K, K)
            return (pltpu.make_async_copy(cols_hbm.at[sl], st[0], st[4]),
                    pltpu.make_async_copy(vals_hbm.at[sl], st[1], st[4]),
                    pltpu.make_async_copy(rows_hbm.at[sl], st[2], st[4]))

        def start_idx(k, st):
            for d in idx_copies(k, st):
                d.start()

        def wait_idx(k, st):
            for d in idx_copies(k, st):
                d.wait()

        def gather_copy(st):
            return pltpu.make_async_copy(support_hbm.at[st[0]], st[3], st[5])

        def start_scatter(st):
            pltpu.async_copy(st[3], acc.at[st[2]], st[6], add=True)

        def wait_scatter(st):
            pltpu.make_async_copy(st[3], acc.at[st[2]], st[6]).wait()

        def scale(st):
            gath, vb = st[3], st[1]

            for g in range(K // LANES):
                v16 = vb[pl.ds(g * LANES, LANES)]
                for l in range(LANES):
                    val = jnp.broadcast_to(v16[l], (LANES,))
                    e = g * LANES + l
                    for j in range(D_OUT // LANES):
                        sl = pl.ds(j * LANES, LANES)
                        gath[e, sl] = gath[e, sl] * val

        def step(k, i, drain, prefetch_idx, prefetch_gather):
            cur = sets[i]
            if drain:
                wait_scatter(sets[(i + 2) % NSETS])
            if prefetch_idx:
                start_idx(k + 2, sets[(i + 2) % NSETS])
            if prefetch_gather:
                nxt = sets[(i + 1) % NSETS]
                wait_idx(k + 1, nxt)
                gather_copy(nxt).start()
            gather_copy(cur).wait()
            scale(cur)
            start_scatter(cur)

        start_idx(0, sets[0])
        start_idx(1, sets[1])

        zvec = jnp.zeros((LANES,), jnp.float32)

        def zrow(r, carry):
            for j in range(D_OUT // LANES):
                zbuf[r, pl.ds(j * LANES, LANES)] = zvec
            return carry

        lax.fori_loop(0, K, zrow, 0)
        rbase = s * ROWS_PER_SUB
        for blk in range(ROWS_PER_SUB // K):
            pltpu.sync_copy(zbuf, acc.at[pl.ds(rbase + blk * K, K), :])
        rem = ROWS_PER_SUB % K
        if rem:
            pltpu.sync_copy(
                zbuf.at[pl.ds(0, rem), :],
                acc.at[pl.ds(rbase + (ROWS_PER_SUB // K) * K, rem), :],
            )

        @pl.when(s == NS - 1)
        def _():
            tb = NS * ROWS_PER_SUB
            pltpu.sync_copy(
                zbuf.at[pl.ds(0, ROWS_TAIL), :],
                acc.at[pl.ds(tb, ROWS_TAIL), :],
            )

        wait_idx(0, sets[0])
        gather_copy(sets[0]).start()
        plsc.subcore_barrier()

        step(0, 0, drain=False, prefetch_idx=True, prefetch_gather=True)
        step(1, 1, drain=False, prefetch_idx=True, prefetch_gather=True)

        def quad_body(t, carry):
            k = 4 * t + 2
            for i in range(NSETS):
                step(k + i, (2 + i) % NSETS, drain=True,
                     prefetch_idx=True, prefetch_gather=True)
            return carry

        lax.fori_loop(0, (NCHUNK - 4) // 4, quad_body, 0)

        step(NCHUNK - 2, (NCHUNK - 2) % NSETS, drain=True,
             prefetch_idx=False, prefetch_gather=True)
        step(NCHUNK - 1, (NCHUNK - 1) % NSETS, drain=True,
             prefetch_idx=False, prefetch_gather=False)
        wait_scatter(sets[(NCHUNK - 2) % NSETS])
        wait_scatter(sets[(NCHUNK - 1) % NSETS])

        plsc.subcore_barrier()

        def wb_copy(off, n, sem):
            return pltpu.make_async_copy(
                acc.at[pl.ds(off, n), :], out_hbm.at[c, pl.ds(off, n), :], sem)

        esem = sets[0][4]
        wb_copy(rbase, ROWS_PER_SUB, esem).start()

        @pl.when(s == NS - 1)
        def _():
            wb_copy(NS * ROWS_PER_SUB, ROWS_TAIL, esem).start()

        wb_copy(rbase, ROWS_PER_SUB, esem).wait()

        @pl.when(s == NS - 1)
        def _():
            wb_copy(NS * ROWS_PER_SUB, ROWS_TAIL, esem).wait()

    return spmm(support, cols, vals, rows)


def _matmul(x, W):
    def body(x_ref, w_ref, o_ref):
        o_ref[...] = jnp.dot(x_ref[...], w_ref[...],
                             preferred_element_type=jnp.float32)

    return pl.pallas_call(
        body,
        grid=(10,),
        in_specs=[
            pl.BlockSpec((N_NODES // 10, D_IN), lambda i: (i, 0)),
            pl.BlockSpec((D_IN, D_OUT), lambda i: (0, 0)),
        ],
        out_specs=pl.BlockSpec((N_NODES // 10, D_OUT), lambda i: (i, 0)),
        out_shape=jax.ShapeDtypeStruct((N_NODES, D_OUT), jnp.float32),
    )(x, W)


def _merge(partials, b):
    def body(p_ref, b_ref, o_ref):
        o_ref[...] = p_ref[0] + p_ref[1] + b_ref[...]

    return pl.pallas_call(
        body,
        grid=(10,),
        in_specs=[
            pl.BlockSpec((NC, N_NODES // 10, D_OUT), lambda i: (0, i, 0)),
            pl.BlockSpec((1, D_OUT), lambda i: (0, 0)),
        ],
        out_specs=pl.BlockSpec((N_NODES // 10, D_OUT), lambda i: (i, 0)),
        out_shape=jax.ShapeDtypeStruct((N_NODES, D_OUT), jnp.float32),
    )(partials, b.reshape(1, D_OUT))


def kernel(x, edge_index, edge_values, W, b):
    rows = edge_index[0].astype(jnp.int32)
    cols = edge_index[1].astype(jnp.int32)

    pad = E_PAD - N_EDGES
    padidx = jnp.arange(pad, dtype=jnp.int32) % N_NODES
    cols_p = jnp.concatenate([cols, padidx])
    rows_p = jnp.concatenate([rows, padidx])
    vals_p = jnp.concatenate([edge_values, jnp.zeros((pad,), jnp.float32)])

    support = _matmul(x, W)
    partials = _sc_spmm(support, cols_p, vals_p, rows_p)
    return _merge(partials, b)


# speedup vs baseline: 1.2703x; 1.2703x over previous
import functools

import jax
import jax.numpy as jnp
from jax import lax
from jax.experimental import pallas as pl
from jax.experimental.pallas import tpu as pltpu
from jax.experimental.pallas import tpu_sc as plsc

N_NODES = 10000
N_EDGES = 320000
D_IN = 128
D_OUT = 128

NC = 2
NS = 16
NW = NC * NS
LANES = 16

K = 64
EPW = 10240
NCHUNK = EPW // K
E_PAD = EPW * NW
NSETS = 4
ROWS_PER_SUB = 624
ROWS_TAIL = N_NODES - ROWS_PER_SUB * NS


def _sc_spmm(support, cols, vals, rows):
    mesh = plsc.VectorSubcoreMesh(
        core_axis_name="c", subcore_axis_name="s", num_cores=NC, num_subcores=NS
    )

    scratch = [pltpu.VMEM_SHARED((N_NODES, D_OUT), jnp.float32)]
    for _ in range(NSETS):
        scratch += [
            pltpu.VMEM((K,), jnp.int32),
            pltpu.VMEM((K,), jnp.float32),
            pltpu.VMEM((K,), jnp.int32),
            pltpu.VMEM((K, D_OUT), jnp.float32),
            pltpu.SemaphoreType.DMA,
            pltpu.SemaphoreType.DMA,
            pltpu.SemaphoreType.DMA,
        ]

    @functools.partial(
        pl.kernel,
        mesh=mesh,
        out_type=jax.ShapeDtypeStruct((NC, N_NODES, D_OUT), jnp.float32),
        scratch_types=scratch,
    )
    def spmm(support_hbm, cols_hbm, vals_hbm, rows_hbm, out_hbm, acc, *bufs):
        c = lax.axis_index("c")
        s = lax.axis_index("s")
        wid = s * NC + c
        pbase = wid * NCHUNK
        sets = tuple(bufs[i * 7:(i + 1) * 7] for i in range(NSETS))
        zbuf = sets[NSETS - 1][3]

        def idx_copies(k, st):
            sl = pl.ds((pbase + k) * K, K)
            return (pltpu.make_async_copy(cols_hbm.at[sl], st[0], st[4]),
                    pltpu.make_async_copy(vals_hbm.at[sl], st[1], st[4]),
                    pltpu.make_async_copy(rows_hbm.at[sl], st[2], st[4]))

        def start_idx(k, st):
            for d in idx_copies(k, st):
                d.start()

        def wait_idx(k, st):
            for d in idx_copies(k, st):
                d.wait()

        def gather_copy(st):
            return pltpu.make_async_copy(support_hbm.at[st[0]], st[3], st[5])

        def start_scatter(st):
            pltpu.async_copy(st[3], acc.at[st[2]], st[6], add=True)

        def wait_scatter(st):
            pltpu.make_async_copy(st[3], acc.at[st[2]], st[6]).wait()

        def scale(st):
            gath, vb = st[3], st[1]

            def group_body(g, carry2):
                v16 = vb[pl.ds(g * LANES, LANES)]
                for l in range(LANES):
                    val = jnp.broadcast_to(v16[l], (LANES,))
                    e = g * LANES + l
                    for j in range(D_OUT // LANES):
                        sl = pl.ds(j * LANES, LANES)
                        gath[e, sl] = gath[e, sl] * val
                return carry2

            lax.fori_loop(0, K // LANES, group_body, 0)

        def step(k, i, drain, prefetch_idx, prefetch_gather):
            cur = sets[i]
            if drain:
                wait_scatter(sets[(i + 2) % NSETS])
            if prefetch_idx:
                start_idx(k + 2, sets[(i + 2) % NSETS])
            if prefetch_gather:
                nxt = sets[(i + 1) % NSETS]
                wait_idx(k + 1, nxt)
                gather_copy(nxt).start()
            gather_copy(cur).wait()
            scale(cur)
            start_scatter(cur)

        start_idx(0, sets[0])
        start_idx(1, sets[1])

        zvec = jnp.zeros((LANES,), jnp.float32)

        def zrow(r, carry):
            for j in range(D_OUT // LANES):
                zbuf[r, pl.ds(j * LANES, LANES)] = zvec
            return carry

        lax.fori_loop(0, K, zrow, 0)
        rbase = s * ROWS_PER_SUB
        for blk in range(ROWS_PER_SUB // K):
            pltpu.sync_copy(zbuf, acc.at[pl.ds(rbase + blk * K, K), :])
        rem = ROWS_PER_SUB % K
        if rem:
            pltpu.sync_copy(
                zbuf.at[pl.ds(0, rem), :],
                acc.at[pl.ds(rbase + (ROWS_PER_SUB // K) * K, rem), :],
            )

        @pl.when(s == NS - 1)
        def _():
            tb = NS * ROWS_PER_SUB
            pltpu.sync_copy(
                zbuf.at[pl.ds(0, ROWS_TAIL), :],
                acc.at[pl.ds(tb, ROWS_TAIL), :],
            )

        wait_idx(0, sets[0])
        gather_copy(sets[0]).start()
        plsc.subcore_barrier()

        step(0, 0, drain=False, prefetch_idx=True, prefetch_gather=True)
        step(1, 1, drain=False, prefetch_idx=True, prefetch_gather=True)

        def quad_body(t, carry):
            k = 4 * t + 2
            for i in range(NSETS):
                step(k + i, (2 + i) % NSETS, drain=True,
                     prefetch_idx=True, prefetch_gather=True)
            return carry

        lax.fori_loop(0, (NCHUNK - 4) // 4, quad_body, 0)

        step(NCHUNK - 2, (NCHUNK - 2) % NSETS, drain=True,
             prefetch_idx=False, prefetch_gather=True)
        step(NCHUNK - 1, (NCHUNK - 1) % NSETS, drain=True,
             prefetch_idx=False, prefetch_gather=False)
        wait_scatter(sets[(NCHUNK - 2) % NSETS])
        wait_scatter(sets[(NCHUNK - 1) % NSETS])

        plsc.subcore_barrier()

        def wb_copy(off, n, sem):
            return pltpu.make_async_copy(
                acc.at[pl.ds(off, n), :], out_hbm.at[c, pl.ds(off, n), :], sem)

        esem = sets[0][4]
        wb_copy(rbase, ROWS_PER_SUB, esem).start()

        @pl.when(s == NS - 1)
        def _():
            wb_copy(NS * ROWS_PER_SUB, ROWS_TAIL, esem).start()

        wb_copy(rbase, ROWS_PER_SUB, esem).wait()

        @pl.when(s == NS - 1)
        def _():
            wb_copy(NS * ROWS_PER_SUB, ROWS_TAIL, esem).wait()

    return spmm(support, cols, vals, rows)


def _matmul(x, W):
    def body(x_ref, w_ref, o_ref):
        o_ref[...] = jnp.dot(x_ref[...], w_ref[...],
                             preferred_element_type=jnp.float32)

    return pl.pallas_call(
        body,
        grid=(10,),
        in_specs=[
            pl.BlockSpec((N_NODES // 10, D_IN), lambda i: (i, 0)),
            pl.BlockSpec((D_IN, D_OUT), lambda i: (0, 0)),
        ],
        out_specs=pl.BlockSpec((N_NODES // 10, D_OUT), lambda i: (i, 0)),
        out_shape=jax.ShapeDtypeStruct((N_NODES, D_OUT), jnp.float32),
    )(x, W)


def _merge(partials, b):
    def body(p_ref, b_ref, o_ref):
        o_ref[...] = p_ref[0] + p_ref[1] + b_ref[...]

    return pl.pallas_call(
        body,
        grid=(10,),
        in_specs=[
            pl.BlockSpec((NC, N_NODES // 10, D_OUT), lambda i: (0, i, 0)),
            pl.BlockSpec((1, D_OUT), lambda i: (0, 0)),
        ],
        out_specs=pl.BlockSpec((N_NODES // 10, D_OUT), lambda i: (i, 0)),
        out_shape=jax.ShapeDtypeStruct((N_NODES, D_OUT), jnp.float32),
    )(partials, b.reshape(1, D_OUT))


def kernel(x, edge_index, edge_values, W, b):
    rows = edge_index[0].astype(jnp.int32)
    cols = edge_index[1].astype(jnp.int32)

    pad = E_PAD - N_EDGES
    padidx = jnp.arange(pad, dtype=jnp.int32) % N_NODES
    cols_p = jnp.concatenate([cols, padidx])
    rows_p = jnp.concatenate([rows, padidx])
    vals_p = jnp.concatenate([edge_values, jnp.zeros((pad,), jnp.float32)])

    support = _matmul(x, W)
    partials = _sc_spmm(support, cols_p, vals_p, rows_p)
    return _merge(partials, b)


# speedup vs baseline: 1.3791x; 1.0857x over previous
import functools

import jax
import jax.numpy as jnp
from jax import lax
from jax.experimental import pallas as pl
from jax.experimental.pallas import tpu as pltpu
from jax.experimental.pallas import tpu_sc as plsc

N_NODES = 10000
N_EDGES = 320000
D_IN = 128
D_OUT = 128

NC = 2
NS = 16
NW = NC * NS
LANES = 16

K = 64
EPW = 10240
NCHUNK = EPW // K
E_PAD = EPW * NW
NSETS = 4
ROWS_PER_SUB = 624
ROWS_TAIL = N_NODES - ROWS_PER_SUB * NS


def _sc_spmm(support, cols, vals, rows):
    mesh = plsc.VectorSubcoreMesh(
        core_axis_name="c", subcore_axis_name="s", num_cores=NC, num_subcores=NS
    )

    scratch = [pltpu.VMEM_SHARED((N_NODES, D_OUT), jnp.float32)]
    for _ in range(NSETS):
        scratch += [
            pltpu.VMEM((K,), jnp.int32),
            pltpu.VMEM((K,), jnp.float32),
            pltpu.VMEM((K,), jnp.int32),
            pltpu.VMEM((K, D_OUT), jnp.float32),
            pltpu.SemaphoreType.DMA,
            pltpu.SemaphoreType.DMA,
            pltpu.SemaphoreType.DMA,
            pltpu.SemaphoreType.DMA,
        ]

    @functools.partial(
        pl.kernel,
        mesh=mesh,
        out_type=jax.ShapeDtypeStruct((NC, N_NODES, D_OUT), jnp.float32),
        scratch_types=scratch,
    )
    def spmm(support_hbm, cols_hbm, vals_hbm, rows_hbm, out_hbm, acc, *bufs):
        c = lax.axis_index("c")
        s = lax.axis_index("s")
        wid = s * NC + c
        pbase = wid * NCHUNK
        sets = tuple(bufs[i * 8:(i + 1) * 8] for i in range(NSETS))
        zbuf = sets[NSETS - 1][3]

        def idx_copies(k, st):
            sl = pl.ds((pbase + k) * K, K)
            return (pltpu.make_async_copy(cols_hbm.at[sl], st[0], st[4]),
                    pltpu.make_async_copy(vals_hbm.at[sl], st[1], st[4]))

        def rx_copy(k, st):
            return pltpu.make_async_copy(
                rows_hbm.at[pl.ds((pbase + k) * K, K)], st[2], st[7])

        def start_idx(k, st):
            for d in idx_copies(k, st):
                d.start()

        def wait_idx(k, st):
            for d in idx_copies(k, st):
                d.wait()

        def gather_copy(st):
            return pltpu.make_async_copy(support_hbm.at[st[0]], st[3], st[5])

        def start_scatter(st):
            pltpu.async_copy(st[3], acc.at[st[2]], st[6], add=True)

        def wait_scatter(st):
            pltpu.make_async_copy(st[3], acc.at[st[2]], st[6]).wait()

        def scale(st):
            gath, vb = st[3], st[1]

            def group_body(g, carry2):
                v16 = vb[pl.ds(g * LANES, LANES)]
                for l in range(LANES):
                    val = jnp.broadcast_to(v16[l], (LANES,))
                    e = g * LANES + l
                    for j in range(D_OUT // LANES):
                        sl = pl.ds(j * LANES, LANES)
                        gath[e, sl] = gath[e, sl] * val
                return carry2

            lax.fori_loop(0, K // LANES, group_body, 0)

        def step(k, i, drain=True, pre_rx=True, pre_pb=True, pre_g=True):
            cur = sets[i]
            if drain:
                wait_scatter(sets[(i + 2) % NSETS])
            if pre_rx:
                rx_copy(k + 2, sets[(i + 2) % NSETS]).start()
            if pre_pb:
                start_idx(k + 3, sets[(i + 3) % NSETS])
            if pre_g:
                nn = sets[(i + 2) % NSETS]
                wait_idx(k + 2, nn)
                gather_copy(nn).start()
            gather_copy(cur).wait()
            scale(cur)
            rx_copy(k, cur).wait()
            start_scatter(cur)

        start_idx(0, sets[0])
        start_idx(1, sets[1])
        start_idx(2, sets[2])
        rx_copy(0, sets[0]).start()
        rx_copy(1, sets[1]).start()

        zvec = jnp.zeros((LANES,), jnp.float32)

        def zrow(r, carry):
            for j in range(D_OUT // LANES):
                zbuf[r, pl.ds(j * LANES, LANES)] = zvec
            return carry

        lax.fori_loop(0, K, zrow, 0)
        rbase = s * ROWS_PER_SUB
        for blk in range(ROWS_PER_SUB // K):
            pltpu.sync_copy(zbuf, acc.at[pl.ds(rbase + blk * K, K), :])
        rem = ROWS_PER_SUB % K
        if rem:
            pltpu.sync_copy(
                zbuf.at[pl.ds(0, rem), :],
                acc.at[pl.ds(rbase + (ROWS_PER_SUB // K) * K, rem), :],
            )

        @pl.when(s == NS - 1)
        def _():
            tb = NS * ROWS_PER_SUB
            pltpu.sync_copy(
                zbuf.at[pl.ds(0, ROWS_TAIL), :],
                acc.at[pl.ds(tb, ROWS_TAIL), :],
            )

        wait_idx(0, sets[0])
        gather_copy(sets[0]).start()
        wait_idx(1, sets[1])
        gather_copy(sets[1]).start()
        plsc.subcore_barrier()

        step(0, 0, drain=False)
        step(1, 1, drain=False)

        def quad_body(t, carry):
            k = 4 * t + 2
            for i in range(NSETS):
                step(k + i, (2 + i) % NSETS)
            return carry

        lax.fori_loop(0, (NCHUNK - 6) // 4, quad_body, 0)

        for k in range(4 * ((NCHUNK - 6) // 4) + 2, NCHUNK):
            step(k, k % NSETS,
                 pre_rx=(k + 2 <= NCHUNK - 1),
                 pre_pb=(k + 3 <= NCHUNK - 1),
                 pre_g=(k + 2 <= NCHUNK - 1))
        wait_scatter(sets[(NCHUNK - 2) % NSETS])
        wait_scatter(sets[(NCHUNK - 1) % NSETS])

        plsc.subcore_barrier()

        def wb_copy(off, n, sem):
            return pltpu.make_async_copy(
                acc.at[pl.ds(off, n), :], out_hbm.at[c, pl.ds(off, n), :], sem)

        esem = sets[0][4]
        wb_copy(rbase, ROWS_PER_SUB, esem).start()

        @pl.when(s == NS - 1)
        def _():
            wb_copy(NS * ROWS_PER_SUB, ROWS_TAIL, esem).start()

        wb_copy(rbase, ROWS_PER_SUB, esem).wait()

        @pl.when(s == NS - 1)
        def _():
            wb_copy(NS * ROWS_PER_SUB, ROWS_TAIL, esem).wait()

    return spmm(support, cols, vals, rows)


def _matmul(x, W):
    def body(x_ref, w_ref, o_ref):
        o_ref[...] = jnp.dot(x_ref[...], w_ref[...],
                             preferred_element_type=jnp.float32)

    return pl.pallas_call(
        body,
        grid=(10,),
        in_specs=[
            pl.BlockSpec((N_NODES // 10, D_IN), lambda i: (i, 0)),
            pl.BlockSpec((D_IN, D_OUT), lambda i: (0, 0)),
        ],
        out_specs=pl.BlockSpec((N_NODES // 10, D_OUT), lambda i: (i, 0)),
        out_shape=jax.ShapeDtypeStruct((N_NODES, D_OUT), jnp.float32),
    )(x, W)


def _merge(partials, b):
    def body(p_ref, b_ref, o_ref):
        o_ref[...] = p_ref[0] + p_ref[1] + b_ref[...]

    return pl.pallas_call(
        body,
        grid=(10,),
        in_specs=[
            pl.BlockSpec((NC, N_NODES // 10, D_OUT), lambda i: (0, i, 0)),
            pl.BlockSpec((1, D_OUT), lambda i: (0, 0)),
        ],
        out_specs=pl.BlockSpec((N_NODES // 10, D_OUT), lambda i: (i, 0)),
        out_shape=jax.ShapeDtypeStruct((N_NODES, D_OUT), jnp.float32),
    )(partials, b.reshape(1, D_OUT))


def kernel(x, edge_index, edge_values, W, b):
    rows = edge_index[0].astype(jnp.int32)
    cols = edge_index[1].astype(jnp.int32)

    pad = E_PAD - N_EDGES
    padidx = jnp.arange(pad, dtype=jnp.int32) % N_NODES
    cols_p = jnp.concatenate([cols, padidx])
    rows_p = jnp.concatenate([rows, padidx])
    vals_p = jnp.concatenate([edge_values, jnp.zeros((pad,), jnp.float32)])

    support = _matmul(x, W)
    partials = _sc_spmm(support, cols_p, vals_p, rows_p)
    return _merge(partials, b)


# speedup vs baseline: 1.3834x; 1.0031x over previous
import functools

import jax
import jax.numpy as jnp
from jax import lax
from jax.experimental import pallas as pl
from jax.experimental.pallas import tpu as pltpu
from jax.experimental.pallas import tpu_sc as plsc

N_NODES = 10000
N_EDGES = 320000
D_IN = 128
D_OUT = 128

NC = 2
NS = 16
NW = NC * NS
LANES = 16

K = 64
EPW = 10240
NCHUNK = EPW // K
E_PAD = EPW * NW
NSETS = 4
ROWS_PER_SUB = 624
ROWS_TAIL = N_NODES - ROWS_PER_SUB * NS


def _sc_spmm(support, cols, vals, rows):
    mesh = plsc.VectorSubcoreMesh(
        core_axis_name="c", subcore_axis_name="s", num_cores=NC, num_subcores=NS
    )

    scratch = [pltpu.VMEM_SHARED((N_NODES, D_OUT), jnp.float32)]
    for _ in range(NSETS):
        scratch += [
            pltpu.VMEM((K,), jnp.int32),
            pltpu.VMEM((K,), jnp.float32),
            pltpu.VMEM((K,), jnp.int32),
            pltpu.VMEM((K, D_OUT), jnp.float32),
            pltpu.SemaphoreType.DMA,
            pltpu.SemaphoreType.DMA,
            pltpu.SemaphoreType.DMA,
            pltpu.SemaphoreType.DMA,
        ]

    @functools.partial(
        pl.kernel,
        mesh=mesh,
        out_type=jax.ShapeDtypeStruct((NC, N_NODES, D_OUT), jnp.float32),
        scratch_types=scratch,
    )
    def spmm(support_hbm, cols_hbm, vals_hbm, rows_hbm, out_hbm, acc, *bufs):
        c = lax.axis_index("c")
        s = lax.axis_index("s")
        wid = s * NC + c
        pbase = wid * NCHUNK
        sets = tuple(bufs[i * 8:(i + 1) * 8] for i in range(NSETS))
        zbuf = sets[NSETS - 1][3]

        def idx_copies(k, st):
            sl = pl.ds((pbase + k) * K, K)
            return (pltpu.make_async_copy(cols_hbm.at[sl], st[0], st[4]),
                    pltpu.make_async_copy(vals_hbm.at[sl], st[1], st[4]))

        def rx_copy(k, st):
            return pltpu.make_async_copy(
                rows_hbm.at[pl.ds((pbase + k) * K, K)], st[2], st[7])

        def start_idx(k, st):
            for d in idx_copies(k, st):
                d.start()

        def wait_idx(k, st):
            for d in idx_copies(k, st):
                d.wait()

        def gather_copy(st):
            return pltpu.make_async_copy(support_hbm.at[st[0]], st[3], st[5])

        def start_scatter(st):
            pltpu.async_copy(st[3], acc.at[st[2]], st[6], add=True)

        def wait_scatter(st):
            pltpu.make_async_copy(st[3], acc.at[st[2]], st[6]).wait()

        def scale(st):
            gath, vb = st[3], st[1]

            def group_body(g, carry2):
                v16 = vb[pl.ds(g * LANES, LANES)]
                for l in range(LANES):
                    val = jnp.broadcast_to(v16[l], (LANES,))
                    e = g * LANES + l
                    for j in range(D_OUT // LANES):
                        sl = pl.ds(j * LANES, LANES)
                        gath[e, sl] = gath[e, sl] * val
                return carry2

            lax.fori_loop(0, K // LANES, group_body, 0)

        def step(k, i, drain=True, pre_rx=True, pre_pb=True, pre_g=True):
            cur = sets[i]
            if drain:
                wait_scatter(sets[(i + 2) % NSETS])
            if pre_rx:
                rx_copy(k + 2, sets[(i + 2) % NSETS]).start()
            if pre_pb:
                start_idx(k + 3, sets[(i + 3) % NSETS])
            if pre_g:
                nn = sets[(i + 2) % NSETS]
                wait_idx(k + 2, nn)
                gather_copy(nn).start()
            gather_copy(cur).wait()
            scale(cur)
            rx_copy(k, cur).wait()
            start_scatter(cur)

        start_idx(0, sets[0])
        start_idx(1, sets[1])
        start_idx(2, sets[2])
        rx_copy(0, sets[0]).start()
        rx_copy(1, sets[1]).start()

        zvec = jnp.zeros((LANES,), jnp.float32)

        def zrow(r, carry):
            for j in range(D_OUT // LANES):
                zbuf[r, pl.ds(j * LANES, LANES)] = zvec
            return carry

        lax.fori_loop(0, K, zrow, 0)
        rbase = s * ROWS_PER_SUB
        zsem = sets[NSETS - 1][5]

        def zcopy(off, n):
            return pltpu.make_async_copy(
                zbuf.at[pl.ds(0, n), :], acc.at[pl.ds(off, n), :], zsem)

        rem = ROWS_PER_SUB % K
        for blk in range(ROWS_PER_SUB // K):
            zcopy(rbase + blk * K, K).start()
        if rem:
            zcopy(rbase + (ROWS_PER_SUB // K) * K, rem).start()

        @pl.when(s == NS - 1)
        def _():
            zcopy(NS * ROWS_PER_SUB, ROWS_TAIL).start()

        for blk in range(ROWS_PER_SUB // K):
            zcopy(rbase + blk * K, K).wait()
        if rem:
            zcopy(rbase + (ROWS_PER_SUB // K) * K, rem).wait()

        @pl.when(s == NS - 1)
        def _():
            zcopy(NS * ROWS_PER_SUB, ROWS_TAIL).wait()

        wait_idx(0, sets[0])
        gather_copy(sets[0]).start()
        wait_idx(1, sets[1])
        gather_copy(sets[1]).start()
        plsc.subcore_barrier()

        step(0, 0, drain=False)
        step(1, 1, drain=False)

        def quad_body(t, carry):
            k = 4 * t + 2
            for i in range(NSETS):
                step(k + i, (2 + i) % NSETS)
            return carry

        lax.fori_loop(0, (NCHUNK - 6) // 4, quad_body, 0)

        for k in range(4 * ((NCHUNK - 6) // 4) + 2, NCHUNK):
            step(k, k % NSETS,
                 pre_rx=(k + 2 <= NCHUNK - 1),
                 pre_pb=(k + 3 <= NCHUNK - 1),
                 pre_g=(k + 2 <= NCHUNK - 1))
        wait_scatter(sets[(NCHUNK - 2) % NSETS])
        wait_scatter(sets[(NCHUNK - 1) % NSETS])

        plsc.subcore_barrier()

        def wb_copy(off, n, sem):
            return pltpu.make_async_copy(
                acc.at[pl.ds(off, n), :], out_hbm.at[c, pl.ds(off, n), :], sem)

        esem = sets[0][4]
        wb_copy(rbase, ROWS_PER_SUB, esem).start()

        @pl.when(s == NS - 1)
        def _():
            wb_copy(NS * ROWS_PER_SUB, ROWS_TAIL, esem).start()

        wb_copy(rbase, ROWS_PER_SUB, esem).wait()

        @pl.when(s == NS - 1)
        def _():
            wb_copy(NS * ROWS_PER_SUB, ROWS_TAIL, esem).wait()

    return spmm(support, cols, vals, rows)


def _matmul(x, W):
    def body(x_ref, w_ref, o_ref):
        o_ref[...] = jnp.dot(x_ref[...], w_ref[...],
                             preferred_element_type=jnp.float32)

    return pl.pallas_call(
        body,
        grid=(10,),
        in_specs=[
            pl.BlockSpec((N_NODES // 10, D_IN), lambda i: (i, 0)),
            pl.BlockSpec((D_IN, D_OUT), lambda i: (0, 0)),
        ],
        out_specs=pl.BlockSpec((N_NODES // 10, D_OUT), lambda i: (i, 0)),
        out_shape=jax.ShapeDtypeStruct((N_NODES, D_OUT), jnp.float32),
    )(x, W)


def _merge(partials, b):
    def body(p_ref, b_ref, o_ref):
        o_ref[...] = p_ref[0] + p_ref[1] + b_ref[...]

    return pl.pallas_call(
        body,
        grid=(10,),
        in_specs=[
            pl.BlockSpec((NC, N_NODES // 10, D_OUT), lambda i: (0, i, 0)),
            pl.BlockSpec((1, D_OUT), lambda i: (0, 0)),
        ],
        out_specs=pl.BlockSpec((N_NODES // 10, D_OUT), lambda i: (i, 0)),
        out_shape=jax.ShapeDtypeStruct((N_NODES, D_OUT), jnp.float32),
    )(partials, b.reshape(1, D_OUT))


def kernel(x, edge_index, edge_values, W, b):
    rows = edge_index[0].astype(jnp.int32)
    cols = edge_index[1].astype(jnp.int32)

    pad = E_PAD - N_EDGES
    padidx = jnp.arange(pad, dtype=jnp.int32) % N_NODES
    cols_p = jnp.concatenate([cols, padidx])
    rows_p = jnp.concatenate([rows, padidx])
    vals_p = jnp.concatenate([edge_values, jnp.zeros((pad,), jnp.float32)])

    support = _matmul(x, W)
    partials = _sc_spmm(support, cols_p, vals_p, rows_p)
    return _merge(partials, b)


# speedup vs baseline: 1.4250x; 1.0301x over previous
import functools

import jax
import jax.numpy as jnp
from jax import lax
from jax.experimental import pallas as pl
from jax.experimental.pallas import tpu as pltpu
from jax.experimental.pallas import tpu_sc as plsc

N_NODES = 10000
N_EDGES = 320000
D_IN = 128
D_OUT = 128

NC = 2
NS = 16
NW = NC * NS
LANES = 16

K = 64
EPW = 10240
NCHUNK = EPW // K
E_PAD = EPW * NW
NSETS = 4
ROWS_PER_SUB = 624
ROWS_TAIL = N_NODES - ROWS_PER_SUB * NS


def _sc_spmm(support, cols, vals, rows):
    mesh = plsc.VectorSubcoreMesh(
        core_axis_name="c", subcore_axis_name="s", num_cores=NC, num_subcores=NS
    )

    scratch = [pltpu.VMEM_SHARED((N_NODES, D_OUT), jnp.float32)]
    for _ in range(NSETS):
        scratch += [
            pltpu.VMEM((K,), jnp.int32),
            pltpu.VMEM((K,), jnp.float32),
            pltpu.VMEM((K,), jnp.int32),
            pltpu.VMEM((K, D_OUT), jnp.float32),
            pltpu.SemaphoreType.DMA,
            pltpu.SemaphoreType.DMA,
            pltpu.SemaphoreType.DMA,
            pltpu.SemaphoreType.DMA,
        ]

    @functools.partial(
        pl.kernel,
        mesh=mesh,
        out_type=jax.ShapeDtypeStruct((NC, N_NODES, D_OUT), jnp.float32),
        scratch_types=scratch,
    )
    def spmm(support_hbm, cols_hbm, vals_hbm, rows_hbm, out_hbm, acc, *bufs):
        c = lax.axis_index("c")
        s = lax.axis_index("s")
        wid = s * NC + c
        pbase = wid * NCHUNK
        sets = tuple(bufs[i * 8:(i + 1) * 8] for i in range(NSETS))
        zbuf = sets[NSETS - 1][3]

        def idx_copies(k, st):
            sl = pl.ds((pbase + k) * K, K)
            return (pltpu.make_async_copy(cols_hbm.at[sl], st[0], st[4]),
                    pltpu.make_async_copy(vals_hbm.at[sl], st[1], st[4]))

        def rx_copy(k, st):
            return pltpu.make_async_copy(
                rows_hbm.at[pl.ds((pbase + k) * K, K)], st[2], st[7])

        def start_idx(k, st):
            for d in idx_copies(k, st):
                d.start()

        def wait_idx(k, st):
            for d in idx_copies(k, st):
                d.wait()

        def gather_copy(st):
            return pltpu.make_async_copy(support_hbm.at[st[0]], st[3], st[5])

        def start_scatter(st):
            pltpu.async_copy(st[3], acc.at[st[2]], st[6], add=True)

        def wait_scatter(st):
            pltpu.make_async_copy(st[3], acc.at[st[2]], st[6]).wait()

        def scale(st):
            gath, vb = st[3], st[1]

            def group_body(g, carry2):
                v16 = vb[pl.ds(g * LANES, LANES)]
                for l in range(LANES):
                    val = jnp.broadcast_to(v16[l], (LANES,))
                    e = g * LANES + l
                    for j in range(D_OUT // LANES):
                        sl = pl.ds(j * LANES, LANES)
                        gath[e, sl] = gath[e, sl] * val
                return carry2

            lax.fori_loop(0, K // LANES, group_body, 0)

        def step(k, i, drain=True, pre_rx=True, pre_pb=True, pre_g=True):
            cur = sets[i]
            if drain:
                wait_scatter(sets[(i + 2) % NSETS])
            if pre_rx:
                rx_copy(k + 2, sets[(i + 2) % NSETS]).start()
            if pre_pb:
                start_idx(k + 3, sets[(i + 3) % NSETS])
            if pre_g:
                nn = sets[(i + 2) % NSETS]
                wait_idx(k + 2, nn)
                gather_copy(nn).start()
            gather_copy(cur).wait()
            scale(cur)
            rx_copy(k, cur).wait()
            start_scatter(cur)

        start_idx(0, sets[0])
        start_idx(1, sets[1])
        start_idx(2, sets[2])
        rx_copy(0, sets[0]).start()
        rx_copy(1, sets[1]).start()

        zvec = jnp.zeros((LANES,), jnp.float32)

        def zrow(r, carry):
            for j in range(D_OUT // LANES):
                zbuf[r, pl.ds(j * LANES, LANES)] = zvec
            return carry

        lax.fori_loop(0, K, zrow, 0)
        rbase = s * ROWS_PER_SUB
        zsem = sets[NSETS - 1][5]

        def zcopy(off, n):
            return pltpu.make_async_copy(
                zbuf.at[pl.ds(0, n), :], acc.at[pl.ds(off, n), :], zsem)

        rem = ROWS_PER_SUB % K
        for blk in range(ROWS_PER_SUB // K):
            zcopy(rbase + blk * K, K).start()
        if rem:
            zcopy(rbase + (ROWS_PER_SUB // K) * K, rem).start()

        @pl.when(s == NS - 1)
        def _():
            zcopy(NS * ROWS_PER_SUB, ROWS_TAIL).start()

        for blk in range(ROWS_PER_SUB // K):
            zcopy(rbase + blk * K, K).wait()
        if rem:
            zcopy(rbase + (ROWS_PER_SUB // K) * K, rem).wait()

        @pl.when(s == NS - 1)
        def _():
            zcopy(NS * ROWS_PER_SUB, ROWS_TAIL).wait()

        wait_idx(0, sets[0])
        gather_copy(sets[0]).start()
        wait_idx(1, sets[1])
        gather_copy(sets[1]).start()
        plsc.subcore_barrier()

        step(0, 0, drain=False)
        step(1, 1, drain=False)

        def quad_body(t, carry):
            k = 4 * t + 2
            for i in range(NSETS):
                step(k + i, (2 + i) % NSETS)
            return carry

        lax.fori_loop(0, (NCHUNK - 6) // 4, quad_body, 0)

        for k in range(4 * ((NCHUNK - 6) // 4) + 2, NCHUNK):
            step(k, k % NSETS,
                 pre_rx=(k + 2 <= NCHUNK - 1),
                 pre_pb=(k + 3 <= NCHUNK - 1),
                 pre_g=(k + 2 <= NCHUNK - 1))
        wait_scatter(sets[(NCHUNK - 2) % NSETS])
        wait_scatter(sets[(NCHUNK - 1) % NSETS])

        plsc.subcore_barrier()

        def wb_copy(off, n, sem):
            return pltpu.make_async_copy(
                acc.at[pl.ds(off, n), :], out_hbm.at[c, pl.ds(off, n), :], sem)

        esem = sets[0][4]
        wb_copy(rbase, ROWS_PER_SUB, esem).start()

        @pl.when(s == NS - 1)
        def _():
            wb_copy(NS * ROWS_PER_SUB, ROWS_TAIL, esem).start()

        wb_copy(rbase, ROWS_PER_SUB, esem).wait()

        @pl.when(s == NS - 1)
        def _():
            wb_copy(NS * ROWS_PER_SUB, ROWS_TAIL, esem).wait()

    return spmm(support, cols, vals, rows)


def _matmul(x, W):
    def body(x_ref, w_ref, o_ref):
        o_ref[...] = jnp.dot(x_ref[...], w_ref[...],
                             preferred_element_type=jnp.float32)

    return pl.pallas_call(
        body,
        grid=(5,),
        in_specs=[
            pl.BlockSpec((N_NODES // 5, D_IN), lambda i: (i, 0)),
            pl.BlockSpec((D_IN, D_OUT), lambda i: (0, 0)),
        ],
        out_specs=pl.BlockSpec((N_NODES // 5, D_OUT), lambda i: (i, 0)),
        out_shape=jax.ShapeDtypeStruct((N_NODES, D_OUT), jnp.float32),
    )(x, W)


def _merge(partials, b):
    def body(p_ref, b_ref, o_ref):
        o_ref[...] = p_ref[0] + p_ref[1] + b_ref[...]

    return pl.pallas_call(
        body,
        grid=(5,),
        in_specs=[
            pl.BlockSpec((NC, N_NODES // 5, D_OUT), lambda i: (0, i, 0)),
            pl.BlockSpec((1, D_OUT), lambda i: (0, 0)),
        ],
        out_specs=pl.BlockSpec((N_NODES // 5, D_OUT), lambda i: (i, 0)),
        out_shape=jax.ShapeDtypeStruct((N_NODES, D_OUT), jnp.float32),
    )(partials, b.reshape(1, D_OUT))


def kernel(x, edge_index, edge_values, W, b):
    rows = edge_index[0].astype(jnp.int32)
    cols = edge_index[1].astype(jnp.int32)

    pad = E_PAD - N_EDGES
    padidx = jnp.arange(pad, dtype=jnp.int32) % N_NODES
    cols_p = jnp.concatenate([cols, padidx])
    rows_p = jnp.concatenate([rows, padidx])
    vals_p = jnp.concatenate([edge_values, jnp.zeros((pad,), jnp.float32)])

    support = _matmul(x, W)
    partials = _sc_spmm(support, cols_p, vals_p, rows_p)
    return _merge(partials, b)
